# baseline (speedup 1.0000x reference)
import jax
import jax.numpy as jnp
from jax import lax
from jax.experimental import pallas as pl
from jax.experimental.pallas import tpu as pltpu
from jax.experimental.pallas import tpu_sc as plsc

VOCAB = 100000
EMBED = 64
CTX = 8
BATCH = 1024

NC = 2
NS = 16
NW = NC * NS
B_PER_W = BATCH // NW
ROWS_PER_W = B_PER_W * CTX
IDX_CHUNK = 128
N_CHUNKS = ROWS_PER_W // IDX_CHUNK

VBLK = 4096


def _sc_gather_sum_body(idx_hbm, table_hbm, out_hbm, idx_v, idx_t, rows_v, emb_v, sem):
    wid = lax.axis_index("s") * NC + lax.axis_index("c")
    base = wid * B_PER_W
    pltpu.sync_copy(idx_hbm.at[:, pl.ds(base, B_PER_W)], idx_v)
    for c in range(CTX):
        for h in range(B_PER_W // 16):
            lanes = lax.iota(jnp.int32, 16) + (h * 16)
            vals = idx_v[c, pl.ds(h * 16, 16)]
            plsc.store_scatter(idx_t, [lanes * CTX + c], vals)
    for j in range(N_CHUNKS):
        pltpu.async_copy(
            table_hbm.at[idx_t.at[pl.ds(j * IDX_CHUNK, IDX_CHUNK)]],
            rows_v.at[pl.ds(j * IDX_CHUNK, IDX_CHUNK)],
            sem,
        ).wait()

    zeros16 = jnp.zeros((16,), jnp.float32)

    def body(lb, carry):
        r0 = lb * CTX
        for d in range(EMBED // 16):
            col = pl.ds(d * 16, 16)
            acc = rows_v[r0, col]
            for c in range(1, CTX):
                acc = acc + rows_v[r0 + c, col]
            emb_v[lb, col] = acc
        for d in range(EMBED // 16):
            emb_v[lb, pl.ds(EMBED + d * 16, 16)] = zeros16
        return carry

    lax.fori_loop(0, B_PER_W, body, 0)
    pltpu.sync_copy(emb_v, out_hbm.at[pl.ds(base, B_PER_W)])


@jax.jit
def _sc_gather_sum(idx, table):
    mesh = plsc.VectorSubcoreMesh(core_axis_name="c", subcore_axis_name="s")
    return pl.kernel(
        _sc_gather_sum_body,
        out_type=jax.ShapeDtypeStruct((BATCH, 2 * EMBED), jnp.float32),
        mesh=mesh,
        scratch_types=[
            pltpu.VMEM((CTX, B_PER_W), jnp.int32),
            pltpu.VMEM((ROWS_PER_W,), jnp.int32),
            pltpu.VMEM((ROWS_PER_W, EMBED), jnp.float32),
            pltpu.VMEM((B_PER_W, 2 * EMBED), jnp.float32),
            pltpu.SemaphoreType.DMA,
        ],
        compiler_params=pltpu.CompilerParams(use_tc_tiling_on_sc=False, needs_layout_passes=False),
    )(idx, table)


def _mm_body(w_ref, emb_ref, b_ref, out_ref):
    emb64 = emb_ref[...][:, :EMBED]
    out_ref[...] = (
        lax.dot_general(
            w_ref[...],
            emb64,
            (((1,), (1,)), ((), ())),
            preferred_element_type=jnp.float32,
        )
        + b_ref[...]
    )


@jax.jit
def _tc_project(embeds, W, b2d):
    grid = (pl.cdiv(VOCAB, VBLK),)
    return pl.pallas_call(
        _mm_body,
        grid=grid,
        in_specs=[
            pl.BlockSpec((VBLK, EMBED), lambda i: (i, 0)),
            pl.BlockSpec((BATCH, 2 * EMBED), lambda i: (0, 0)),
            pl.BlockSpec((VBLK, 1), lambda i: (i, 0)),
        ],
        out_specs=pl.BlockSpec((VBLK, BATCH), lambda i: (i, 0)),
        out_shape=jax.ShapeDtypeStruct((VOCAB, BATCH), jnp.float32),
        compiler_params=pltpu.CompilerParams(
            dimension_semantics=("parallel",),
        ),
    )(W, embeds, b2d)


def kernel(inputs, emb_table, W, b):
    embeds = _sc_gather_sum(inputs.astype(jnp.int32), emb_table)
    return _tc_project(embeds, W, b.reshape(VOCAB, 1)).T


# speedup vs baseline: 1.0014x; 1.0014x over previous
import jax
import jax.numpy as jnp
from jax import lax
from jax.experimental import pallas as pl
from jax.experimental.pallas import tpu as pltpu
from jax.experimental.pallas import tpu_sc as plsc

VOCAB = 100000
EMBED = 64
CTX = 8
BATCH = 1024

NC = 2
NS = 16
NW = NC * NS
B_PER_W = BATCH // NW
ROWS_PER_W = B_PER_W * CTX
IDX_CHUNK = 128
N_CHUNKS = ROWS_PER_W // IDX_CHUNK

VBLK = 5632


def _sc_gather_sum_body(idx_hbm, table_hbm, out_hbm, idx_v, rows_v, emb_v, sem):
    wid = lax.axis_index("s") * NC + lax.axis_index("c")
    base = wid * B_PER_W
    pltpu.sync_copy(idx_hbm.at[:, pl.ds(base, B_PER_W)], idx_v)
    copies = [
        pltpu.async_copy(
            table_hbm.at[idx_v.at[c]],
            rows_v.at[pl.ds(c * B_PER_W, B_PER_W)],
            sem,
        )
        for c in range(CTX)
    ]
    for cp in copies:
        cp.wait()

    zeros16 = jnp.zeros((16,), jnp.float32)

    def body(lb, carry):
        for d in range(EMBED // 16):
            col = pl.ds(d * 16, 16)
            acc = rows_v[lb, col]
            for c in range(1, CTX):
                acc = acc + rows_v[c * B_PER_W + lb, col]
            emb_v[lb, col] = acc
        for d in range(EMBED // 16):
            emb_v[lb, pl.ds(EMBED + d * 16, 16)] = zeros16
        return carry

    lax.fori_loop(0, B_PER_W, body, 0)
    pltpu.sync_copy(emb_v, out_hbm.at[pl.ds(base, B_PER_W)])


@jax.jit
def _sc_gather_sum(idx, table):
    mesh = plsc.VectorSubcoreMesh(core_axis_name="c", subcore_axis_name="s")
    return pl.kernel(
        _sc_gather_sum_body,
        out_type=jax.ShapeDtypeStruct((BATCH, 2 * EMBED), jnp.float32),
        mesh=mesh,
        scratch_types=[
            pltpu.VMEM((CTX, B_PER_W), jnp.int32),
            pltpu.VMEM((ROWS_PER_W, EMBED), jnp.float32),
            pltpu.VMEM((B_PER_W, 2 * EMBED), jnp.float32),
            pltpu.SemaphoreType.DMA,
        ],
        compiler_params=pltpu.CompilerParams(use_tc_tiling_on_sc=False),
    )(idx, table)


def _mm_body(w_ref, emb_ref, b_ref, out_ref):
    emb64 = emb_ref[...][:, :EMBED]
    out_ref[...] = (
        lax.dot_general(
            w_ref[...],
            emb64,
            (((1,), (1,)), ((), ())),
            preferred_element_type=jnp.float32,
        )
        + b_ref[...]
    )


@jax.jit
def _tc_project(embeds, W, b2d):
    grid = (pl.cdiv(VOCAB, VBLK),)
    return pl.pallas_call(
        _mm_body,
        grid=grid,
        in_specs=[
            pl.BlockSpec((VBLK, EMBED), lambda i: (i, 0)),
            pl.BlockSpec((BATCH, 2 * EMBED), lambda i: (0, 0)),
            pl.BlockSpec((VBLK, 1), lambda i: (i, 0)),
        ],
        out_specs=pl.BlockSpec((VBLK, BATCH), lambda i: (i, 0)),
        out_shape=jax.ShapeDtypeStruct((VOCAB, BATCH), jnp.float32),
        compiler_params=pltpu.CompilerParams(
            dimension_semantics=("parallel",),
        ),
    )(W, embeds, b2d)


def kernel(inputs, emb_table, W, b):
    embeds = _sc_gather_sum(inputs.astype(jnp.int32), emb_table)
    return _tc_project(embeds, W, b.reshape(VOCAB, 1)).T


# speedup vs baseline: 1.0107x; 1.0093x over previous
import jax
import jax.numpy as jnp
from jax import lax
from jax.experimental import pallas as pl
from jax.experimental.pallas import tpu as pltpu
from jax.experimental.pallas import tpu_sc as plsc

VOCAB = 100000
EMBED = 64
CTX = 8
BATCH = 1024

NC = 2
NS = 16
NW = NC * NS
B_PER_W = BATCH // NW
ROWS_PER_W = B_PER_W * CTX
IDX_CHUNK = 128
N_CHUNKS = ROWS_PER_W // IDX_CHUNK

VBLK = 5632


def _sc_gather_sum_body(idx_hbm, table_hbm, out_hbm, idx_v, rows_v, emb_v, sem):
    wid = lax.axis_index("s") * NC + lax.axis_index("c")
    base = wid * B_PER_W
    pltpu.sync_copy(idx_hbm.at[:, pl.ds(base, B_PER_W)], idx_v)
    copies = [
        pltpu.async_copy(
            table_hbm.at[idx_v.at[c]],
            rows_v.at[pl.ds(c * B_PER_W, B_PER_W)],
            sem,
        )
        for c in range(CTX)
    ]
    for cp in copies:
        cp.wait()

    zeros16 = jnp.zeros((16,), jnp.float32)

    def body(lb, carry):
        for d in range(EMBED // 16):
            col = pl.ds(d * 16, 16)
            acc = rows_v[lb, col]
            for c in range(1, CTX):
                acc = acc + rows_v[c * B_PER_W + lb, col]
            emb_v[lb, col] = acc
        for d in range(EMBED // 16):
            emb_v[lb, pl.ds(EMBED + d * 16, 16)] = zeros16
        return carry

    lax.fori_loop(0, B_PER_W, body, 0)
    pltpu.sync_copy(emb_v, out_hbm.at[pl.ds(base, B_PER_W)])


@jax.jit
def _sc_gather_sum(idx, table):
    mesh = plsc.VectorSubcoreMesh(core_axis_name="c", subcore_axis_name="s")
    return pl.kernel(
        _sc_gather_sum_body,
        out_type=jax.ShapeDtypeStruct((BATCH, 2 * EMBED), jnp.float32),
        mesh=mesh,
        scratch_types=[
            pltpu.VMEM((CTX, B_PER_W), jnp.int32),
            pltpu.VMEM((ROWS_PER_W, EMBED), jnp.float32),
            pltpu.VMEM((B_PER_W, 2 * EMBED), jnp.float32),
            pltpu.SemaphoreType.DMA,
        ],
        compiler_params=pltpu.CompilerParams(use_tc_tiling_on_sc=False),
    )(idx, table)


def _mm_body(w_ref, emb_ref, b_ref, out_ref):
    emb64 = emb_ref[...][:, :EMBED]
    out_ref[...] = (
        lax.dot_general(
            w_ref[...],
            emb64,
            (((1,), (1,)), ((), ())),
            preferred_element_type=jnp.float32,
        )
        + b_ref[...]
    )


@jax.jit
def _tc_project(embeds, W, b2d):
    grid = (pl.cdiv(VOCAB, VBLK),)
    return pl.pallas_call(
        _mm_body,
        grid=grid,
        in_specs=[
            pl.BlockSpec((VBLK, EMBED), lambda i: (i, 0)),
            pl.BlockSpec((BATCH, 2 * EMBED), lambda i: (0, 0)),
            pl.BlockSpec((VBLK, 1), lambda i: (i, 0)),
        ],
        out_specs=pl.BlockSpec((VBLK, BATCH), lambda i: (i, 0)),
        out_shape=jax.ShapeDtypeStruct((VOCAB, BATCH), jnp.float32),
        compiler_params=pltpu.CompilerParams(
            dimension_semantics=("parallel",),
            allow_input_fusion=(True, True, True),
        ),
    )(W, embeds, b2d)


def kernel(inputs, emb_table, W, b):
    embeds = _sc_gather_sum(inputs.astype(jnp.int32), emb_table)
    return _tc_project(embeds, W, b.reshape(VOCAB, 1)).T
